# baseline (speedup 1.0000x reference)
import functools

import numpy as np
import jax
import jax.numpy as jnp
from jax import lax
from jax.experimental import pallas as pl
from jax.experimental.pallas import tpu as pltpu
from jax.experimental.pallas import tpu_sc as plsc

F = 26
NONE = 21
NMUL = 5
V1 = 100000
VM = 100001
EMB = 16
MAXN = 20
B = 4096
PAIRS = F * (F - 1) // 2
D1 = 128
D2 = 128
BN_EPS = 1e-5

NW = 32
BPW = B // NW

ROWW = F * EMB + EMB
W1C = F * EMB
NCH = ROWW // EMB

ONE0, MUL0, IDXW = 0, 24, 128

_iu, _ju = np.triu_indices(F, 1)
_my_pairs = [(int(a), int(b)) for a, b in zip(_iu, _ju)]
assert len(_my_pairs) == PAIRS


def _sc_gather(w2oneT, w2mulT, idx_all, rinv):
    mesh = plsc.VectorSubcoreMesh(core_axis_name="c", subcore_axis_name="s")

    @functools.partial(
        pl.kernel,
        out_type=[
            jax.ShapeDtypeStruct((B, PAIRS, EMB), jnp.float32),
            jax.ShapeDtypeStruct((B, EMB), jnp.float32),
        ],
        mesh=mesh,
        scratch_types=[
            pltpu.VMEM((IDXW,), jnp.int32),
            pltpu.VMEM((IDXW,), jnp.int32),
            pltpu.VMEM((NMUL, EMB), jnp.float32),
            pltpu.VMEM((NMUL, EMB), jnp.float32),
            pltpu.VMEM((NONE, ROWW), jnp.float32),
            pltpu.VMEM((NONE, ROWW), jnp.float32),
            pltpu.VMEM((MAXN * NMUL, ROWW), jnp.float32),
            pltpu.VMEM((MAXN * NMUL, ROWW), jnp.float32),
            pltpu.VMEM((NMUL * NCH, EMB), jnp.float32),
            pltpu.VMEM((PAIRS, EMB), jnp.float32),
            pltpu.VMEM((PAIRS, EMB), jnp.float32),
            pltpu.VMEM((BPW, EMB), jnp.float32),
            pltpu.SemaphoreType.DMA,
            pltpu.SemaphoreType.DMA,
            pltpu.SemaphoreType.DMA,
            pltpu.SemaphoreType.DMA,
            pltpu.SemaphoreType.DMA,
            pltpu.SemaphoreType.DMA,
            pltpu.SemaphoreType.DMA,
            pltpu.SemaphoreType.DMA,
        ],
        compiler_params=pltpu.CompilerParams(use_tc_tiling_on_sc=False),
    )
    def k(w2oneT_r, w2mulT_r, idx_r, rinv_r, w_out, fo_out,
          idx_v0, idx_v1, rinv_v0, rinv_v1, ot_v0, ot_v1, mt_v0, mt_v1,
          macc_v, wij_v0, wij_v1, fov_v,
          isem0, isem1, got0, got1, gmt0, gmt1, osem0, osem1):
        idx_v = (idx_v0, idx_v1)
        rinv_v = (rinv_v0, rinv_v1)
        ot_v = (ot_v0, ot_v1)
        mt_v = (mt_v0, mt_v1)
        wij_v = (wij_v0, wij_v1)
        isem = (isem0, isem1)
        got = (got0, got1)
        gmt = (gmt0, gmt1)
        osem = (osem0, osem1)

        wid = lax.axis_index("c") * 16 + lax.axis_index("s")
        base = wid * BPW

        def in_descs(buf, row):
            return (
                pltpu.make_async_copy(idx_r.at[row], idx_v[buf], isem[buf]),
                pltpu.make_async_copy(rinv_r.at[row], rinv_v[buf], isem[buf]),
            )

        def gather_descs(buf):
            return (
                pltpu.make_async_copy(
                    w2mulT_r.at[idx_v[buf].at[pl.ds(MUL0, NMUL * MAXN)]],
                    mt_v[buf], gmt[buf]),
                pltpu.make_async_copy(
                    w2oneT_r.at[idx_v[buf].at[pl.ds(ONE0, NONE)]],
                    ot_v[buf], got[buf]),
            )

        def out_desc(buf, row):
            return pltpu.make_async_copy(wij_v[buf], w_out.at[row], osem[buf])

        def reduce_row(buf, e):
            @pl.loop(0, NCH)
            def _chunk(j):
                c0 = j * EMB
                for m in range(NMUL):
                    r0 = m * MAXN
                    acc = mt_v[buf][pl.ds(r0, 1), pl.ds(c0, EMB)]
                    for kk in range(1, MAXN):
                        acc = acc + mt_v[buf][pl.ds(r0 + kk, 1),
                                              pl.ds(c0, EMB)]
                    macc_v[pl.ds(m * NCH + j, 1), :] = (
                        acc * rinv_v[buf][pl.ds(m, 1), :])

            for p, (i, j) in enumerate(_my_pairs):
                if j < NONE:
                    a = ot_v[buf][pl.ds(j, 1), pl.ds(i * EMB, EMB)]
                    bb = ot_v[buf][pl.ds(i, 1), pl.ds(j * EMB, EMB)]
                elif i < NONE:
                    m = j - NONE
                    a = macc_v[pl.ds(m * NCH + i, 1), :]
                    bb = ot_v[buf][pl.ds(i, 1), pl.ds(j * EMB, EMB)]
                else:
                    ma, mb = i - NONE, j - NONE
                    a = macc_v[pl.ds(mb * NCH + i, 1), :]
                    bb = macc_v[pl.ds(ma * NCH + j, 1), :]
                wij_v[buf][pl.ds(p, 1), :] = a * bb

            facc = ot_v[buf][pl.ds(0, 1), pl.ds(W1C, EMB)]
            for f in range(1, NONE):
                facc = facc + ot_v[buf][pl.ds(f, 1), pl.ds(W1C, EMB)]
            for m in range(NMUL):
                facc = facc + macc_v[pl.ds(m * NCH + NCH - 1, 1), :]
            fov_v[pl.ds(e, 1), :] = facc

        for d in in_descs(0, base):
            d.start()
        for d in in_descs(0, base):
            d.wait()
        for d in gather_descs(0):
            d.start()
        for d in in_descs(1, base + 1):
            d.start()

        @pl.loop(0, BPW // 2)
        def _pair_loop(g):
            for h in (0, 1):
                e = 2 * g + h
                bi = base + e
                buf = h
                if h == 0:
                    for d in in_descs(1 - buf, base + e + 1):
                        d.wait()

                    @pl.when(e > 0)
                    def _():
                        out_desc(1 - buf, bi - 1).wait()

                    for d in gather_descs(1 - buf):
                        d.start()
                else:
                    for d in in_descs(1 - buf,
                                      base + lax.min(e + 1, BPW - 1)):
                        d.wait()
                    out_desc(1 - buf, bi - 1).wait()

                    @pl.when(e + 1 < BPW)
                    def _():
                        for d in gather_descs(1 - buf):
                            d.start()

                descs = gather_descs(buf)
                descs[0].wait()
                descs[1].wait()
                pre = base + lax.min(e + 2, BPW - 1)
                in_descs(buf, pre)[0].start()
                reduce_row(buf, e)
                in_descs(buf, pre)[1].start()
                out_desc(buf, bi).start()

        for d in in_descs(1, base):
            d.wait()
        out_desc(1, base).wait()
        pltpu.sync_copy(fov_v, fo_out.at[pl.ds(base, BPW)])

    return k(w2oneT, w2mulT, idx_all, rinv)


def _tc_body(d_ref, fo_ref, w1_ref, c1_ref, w2_ref, c2_ref, bias_ref, o_ref):
    d = d_ref[...]
    x1 = jnp.dot(d, w1_ref[...], preferred_element_type=jnp.float32)
    x1 = jnp.maximum(x1 + c1_ref[...], 0.0)
    x2 = jnp.dot(x1, w2_ref[...], preferred_element_type=jnp.float32)
    x2 = jnp.maximum(x2 + c2_ref[...], 0.0)
    tot = bias_ref[0] + fo_ref[...].sum(axis=1) + d.sum(axis=1) + x2.sum(axis=1)
    o_ref[...] = tot


def _tc_mlp(d2, fov, w1f, c1, w2f, c2, bias):
    blk = 128
    nblk = B // blk
    out = pl.pallas_call(
        _tc_body,
        grid=(nblk,),
        in_specs=[
            pl.BlockSpec((blk, PAIRS * EMB), lambda i: (i, 0)),
            pl.BlockSpec((blk, EMB), lambda i: (i, 0)),
            pl.BlockSpec((PAIRS * EMB, D1), lambda i: (0, 0)),
            pl.BlockSpec((1, D1), lambda i: (0, 0)),
            pl.BlockSpec((D1, D2), lambda i: (0, 0)),
            pl.BlockSpec((1, D2), lambda i: (0, 0)),
            pl.BlockSpec(memory_space=pltpu.SMEM),
        ],
        out_specs=pl.BlockSpec((blk,), lambda i: (i,)),
        out_shape=jax.ShapeDtypeStruct((B,), jnp.float32),
    )(d2, fov, w1f, c1, w2f, c2, bias)
    return out


def kernel(Xi_one, Xi_mul, Xi_mle, bias, W1_one, W1_mul, W2_one, W2_mul,
           Wl1, bl1, Wl2, bl2, bn1_g, bn1_b, bn2_g, bn2_b):
    Xi_one = Xi_one.astype(jnp.int32)
    Xi_mul = Xi_mul.astype(jnp.int32)
    Xi_mle = Xi_mle.astype(jnp.int32)

    w2oneT = jnp.concatenate(
        [jnp.transpose(W2_one, (1, 0, 2)).reshape(V1, F * EMB), W1_one,
         jnp.zeros((V1, EMB - 1), jnp.float32)], axis=1)
    w2mulT = jnp.concatenate(
        [jnp.transpose(W2_mul, (1, 0, 2)).reshape(VM, F * EMB), W1_mul,
         jnp.zeros((VM, EMB - 1), jnp.float32)], axis=1)

    idx_all = jnp.concatenate(
        [Xi_one, jnp.zeros((B, MUL0 - NONE), jnp.int32),
         Xi_mul.reshape(B, NMUL * MAXN),
         jnp.zeros((B, IDXW - MUL0 - NMUL * MAXN), jnp.int32)], axis=1)
    rinv16 = jnp.broadcast_to(
        (1.0 / jnp.maximum(Xi_mle.astype(jnp.float32), 1.0))[:, :, None],
        (B, NMUL, EMB))
    rinv16 = jnp.asarray(rinv16)

    inv = 1.0 / np.sqrt(1.0 + BN_EPS)
    s1 = bn1_g * inv
    w1f = Wl1 * s1[None, :]
    c1 = (bl1 * s1 + bn1_b).reshape(1, D1)
    s2 = bn2_g * inv
    w2f = Wl2 * s2[None, :]
    c2 = (bl2 * s2 + bn2_b).reshape(1, D2)

    w_out, fo_out = _sc_gather(w2oneT, w2mulT, idx_all, rinv16)
    d2 = w_out.reshape(B, PAIRS * EMB)
    return _tc_mlp(d2, fo_out, w1f, c1, w2f, c2, bias)


# speedup vs baseline: 1.4176x; 1.4176x over previous
import functools

import numpy as np
import jax
import jax.numpy as jnp
from jax import lax
from jax.experimental import pallas as pl
from jax.experimental.pallas import tpu as pltpu
from jax.experimental.pallas import tpu_sc as plsc

F = 26
NONE = 21
NMUL = 5
V1 = 100000
VM = 100001
EMB = 16
MAXN = 20
B = 4096
PAIRS = F * (F - 1) // 2
D1 = 128
D2 = 128
BN_EPS = 1e-5

NW = 32
BPW = B // NW

ROWW = F * EMB
NCH = ROWW // EMB
NFO = NONE + NMUL * MAXN

ONE0, MUL0, FO0, IDXW = 0, 24, 128, 256

_iu, _ju = np.triu_indices(F, 1)
_my_pairs = [(int(a), int(b)) for a, b in zip(_iu, _ju)]
assert len(_my_pairs) == PAIRS


def _sc_gather(w2oneT, w2mulT, w1cat, idx_all, rinv):
    mesh = plsc.VectorSubcoreMesh(core_axis_name="c", subcore_axis_name="s")

    @functools.partial(
        pl.kernel,
        out_type=[
            jax.ShapeDtypeStruct((B, PAIRS * EMB), jnp.float32),
            jax.ShapeDtypeStruct((B, EMB), jnp.float32),
        ],
        mesh=mesh,
        scratch_types=[
            pltpu.VMEM((IDXW,), jnp.int32),
            pltpu.VMEM((IDXW,), jnp.int32),
            pltpu.VMEM((NMUL, EMB), jnp.float32),
            pltpu.VMEM((NMUL, EMB), jnp.float32),
            pltpu.VMEM((NONE, ROWW), jnp.float32),
            pltpu.VMEM((NONE, ROWW), jnp.float32),
            pltpu.VMEM((MAXN * NMUL, ROWW), jnp.float32),
            pltpu.VMEM((MAXN * NMUL, ROWW), jnp.float32),
            pltpu.VMEM((NFO, EMB), jnp.float32),
            pltpu.VMEM((NFO, EMB), jnp.float32),
            pltpu.VMEM((NMUL * NCH, EMB), jnp.float32),
            pltpu.VMEM((PAIRS * EMB,), jnp.float32),
            pltpu.VMEM((PAIRS * EMB,), jnp.float32),
            pltpu.VMEM((BPW, EMB), jnp.float32),
            pltpu.SemaphoreType.DMA,
            pltpu.SemaphoreType.DMA,
            pltpu.SemaphoreType.DMA,
            pltpu.SemaphoreType.DMA,
            pltpu.SemaphoreType.DMA,
            pltpu.SemaphoreType.DMA,
            pltpu.SemaphoreType.DMA,
            pltpu.SemaphoreType.DMA,
        ],
        compiler_params=pltpu.CompilerParams(use_tc_tiling_on_sc=False),
    )
    def k(w2oneT_r, w2mulT_r, w1cat_r, idx_r, rinv_r, w_out, fo_out,
          idx_v0, idx_v1, rinv_v0, rinv_v1, ot_v0, ot_v1, mt_v0, mt_v1,
          fo_v0, fo_v1, macc_v, wij_v0, wij_v1, fov_v,
          isem0, isem1, got0, got1, gmt0, gmt1, osem0, osem1):
        idx_v = (idx_v0, idx_v1)
        rinv_v = (rinv_v0, rinv_v1)
        ot_v = (ot_v0, ot_v1)
        fo_v = (fo_v0, fo_v1)
        mt_v = (mt_v0, mt_v1)
        wij_v = (wij_v0, wij_v1)
        isem = (isem0, isem1)
        got = (got0, got1)
        gmt = (gmt0, gmt1)
        osem = (osem0, osem1)

        wid = lax.axis_index("c") * 16 + lax.axis_index("s")
        base = wid * BPW

        def in_descs(buf, row):
            return (
                pltpu.make_async_copy(idx_r.at[row], idx_v[buf], isem[buf]),
                pltpu.make_async_copy(rinv_r.at[row], rinv_v[buf], isem[buf]),
            )

        def gather_descs(buf):
            return (
                pltpu.make_async_copy(
                    w2mulT_r.at[idx_v[buf].at[pl.ds(MUL0, NMUL * MAXN)]],
                    mt_v[buf], gmt[buf]),
                pltpu.make_async_copy(
                    w2oneT_r.at[idx_v[buf].at[pl.ds(ONE0, NONE)]],
                    ot_v[buf], got[buf]),
                pltpu.make_async_copy(
                    w1cat_r.at[idx_v[buf].at[pl.ds(FO0, NFO)]],
                    fo_v[buf], got[buf]),
            )

        def out_desc(buf, row):
            return pltpu.make_async_copy(wij_v[buf], w_out.at[row], osem[buf])

        def reduce_row(buf, e):
            @pl.loop(0, NCH)
            def _chunk(j):
                c0 = j * EMB
                for m in range(NMUL):
                    r0 = m * MAXN
                    acc = mt_v[buf][pl.ds(r0, 1), pl.ds(c0, EMB)]
                    for kk in range(1, MAXN):
                        acc = acc + mt_v[buf][pl.ds(r0 + kk, 1),
                                              pl.ds(c0, EMB)]
                    macc_v[pl.ds(m * NCH + j, 1), :] = (
                        acc * rinv_v[buf][pl.ds(m, 1), :])

            for p, (i, j) in enumerate(_my_pairs):
                if j < NONE:
                    a = ot_v[buf][pl.ds(j, 1), pl.ds(i * EMB, EMB)]
                    bb = ot_v[buf][pl.ds(i, 1), pl.ds(j * EMB, EMB)]
                elif i < NONE:
                    m = j - NONE
                    a = macc_v[pl.ds(m * NCH + i, 1), :]
                    bb = ot_v[buf][pl.ds(i, 1), pl.ds(j * EMB, EMB)]
                else:
                    ma, mb = i - NONE, j - NONE
                    a = macc_v[pl.ds(mb * NCH + i, 1), :]
                    bb = macc_v[pl.ds(ma * NCH + j, 1), :]
                wij_v[buf][pl.ds(p * EMB, EMB)] = (a * bb).reshape((EMB,))

            facc = fo_v[buf][pl.ds(0, 1), :]
            for f in range(1, NONE):
                facc = facc + fo_v[buf][pl.ds(f, 1), :]
            for m in range(NMUL):
                mac = fo_v[buf][pl.ds(NONE + m * MAXN, 1), :]
                for kk in range(1, MAXN):
                    mac = mac + fo_v[buf][pl.ds(NONE + m * MAXN + kk, 1), :]
                facc = facc + mac * rinv_v[buf][pl.ds(m, 1), :]
            fov_v[pl.ds(e, 1), :] = facc

        for d in in_descs(0, base):
            d.start()
        for d in in_descs(0, base):
            d.wait()
        for d in gather_descs(0):
            d.start()
        for d in in_descs(1, base + 1):
            d.start()

        @pl.loop(0, BPW // 2)
        def _pair_loop(g):
            for h in (0, 1):
                e = 2 * g + h
                bi = base + e
                buf = h
                if h == 0:
                    for d in in_descs(1 - buf, base + e + 1):
                        d.wait()

                    @pl.when(e > 0)
                    def _():
                        out_desc(1 - buf, bi - 1).wait()

                    for d in gather_descs(1 - buf):
                        d.start()
                else:
                    for d in in_descs(1 - buf,
                                      base + lax.min(e + 1, BPW - 1)):
                        d.wait()
                    out_desc(1 - buf, bi - 1).wait()

                    @pl.when(e + 1 < BPW)
                    def _():
                        for d in gather_descs(1 - buf):
                            d.start()

                descs = gather_descs(buf)
                descs[0].wait()
                descs[1].wait()
                descs[2].wait()
                pre = base + lax.min(e + 2, BPW - 1)
                in_descs(buf, pre)[0].start()
                reduce_row(buf, e)
                in_descs(buf, pre)[1].start()
                out_desc(buf, bi).start()

        for d in in_descs(1, base):
            d.wait()
        out_desc(1, base).wait()
        pltpu.sync_copy(fov_v, fo_out.at[pl.ds(base, BPW)])

    return k(w2oneT, w2mulT, w1cat, idx_all, rinv)


_TCBLK = 128


def _tc_body(d_hbm, fo_ref, w1_ref, c1_ref, w2_ref, c2_ref, bias_ref, o_ref,
             db_ref, dsem):
    i = pl.program_id(0)
    nblk = pl.num_programs(0)

    def dma(slot, blk_idx):
        return pltpu.make_async_copy(
            d_hbm.at[pl.ds(blk_idx * _TCBLK, _TCBLK), :],
            db_ref.at[slot], dsem.at[slot])

    @pl.when(i == 0)
    def _():
        dma(i % 2, i).start()

    @pl.when(i + 1 < nblk)
    def _():
        dma((i + 1) % 2, i + 1).start()

    dma(i % 2, i).wait()
    d = db_ref[i % 2]
    x1 = jnp.dot(d, w1_ref[...], preferred_element_type=jnp.float32)
    x1 = jnp.maximum(x1 + c1_ref[...], 0.0)
    x2 = jnp.dot(x1, w2_ref[...], preferred_element_type=jnp.float32)
    x2 = jnp.maximum(x2 + c2_ref[...], 0.0)
    tot = bias_ref[0] + fo_ref[...].sum(axis=1) + d.sum(axis=1) + x2.sum(axis=1)
    o_ref[...] = tot


def _tc_mlp(d2, fov, w1f, c1, w2f, c2, bias):
    blk = _TCBLK
    nblk = B // blk
    out = pl.pallas_call(
        _tc_body,
        grid=(nblk,),
        in_specs=[
            pl.BlockSpec(memory_space=pl.ANY),
            pl.BlockSpec((blk, EMB), lambda i: (i, 0)),
            pl.BlockSpec((PAIRS * EMB, D1), lambda i: (0, 0)),
            pl.BlockSpec((1, D1), lambda i: (0, 0)),
            pl.BlockSpec((D1, D2), lambda i: (0, 0)),
            pl.BlockSpec((1, D2), lambda i: (0, 0)),
            pl.BlockSpec(memory_space=pltpu.SMEM),
        ],
        out_specs=pl.BlockSpec((blk,), lambda i: (i,)),
        out_shape=jax.ShapeDtypeStruct((B,), jnp.float32),
        scratch_shapes=[
            pltpu.VMEM((2, blk, PAIRS * EMB), jnp.float32),
            pltpu.SemaphoreType.DMA((2,)),
        ],
    )(d2, fov, w1f, c1, w2f, c2, bias)
    return out


def kernel(Xi_one, Xi_mul, Xi_mle, bias, W1_one, W1_mul, W2_one, W2_mul,
           Wl1, bl1, Wl2, bl2, bn1_g, bn1_b, bn2_g, bn2_b):
    Xi_one = Xi_one.astype(jnp.int32)
    Xi_mul = Xi_mul.astype(jnp.int32)
    Xi_mle = Xi_mle.astype(jnp.int32)

    w2oneT = jnp.transpose(W2_one, (1, 0, 2)).reshape(V1, F * EMB)
    w2mulT = jnp.transpose(W2_mul, (1, 0, 2)).reshape(VM, F * EMB)
    w1flat = jnp.concatenate([W1_one[:, 0], W1_mul[:, 0]])
    w1cat = jnp.pad(w1flat[:, None], ((0, 0), (0, EMB - 1)))

    xim = Xi_mul.reshape(B, NMUL * MAXN)
    idx_all = jnp.concatenate(
        [Xi_one, jnp.zeros((B, MUL0 - NONE), jnp.int32),
         xim, jnp.zeros((B, FO0 - MUL0 - NMUL * MAXN), jnp.int32),
         Xi_one, xim + V1,
         jnp.zeros((B, IDXW - FO0 - NFO), jnp.int32)], axis=1)
    rinv16 = jnp.broadcast_to(
        (1.0 / jnp.maximum(Xi_mle.astype(jnp.float32), 1.0))[:, :, None],
        (B, NMUL, EMB))
    rinv16 = jnp.asarray(rinv16)

    inv = 1.0 / np.sqrt(1.0 + BN_EPS)
    s1 = bn1_g * inv
    w1f = Wl1 * s1[None, :]
    c1 = (bl1 * s1 + bn1_b).reshape(1, D1)
    s2 = bn2_g * inv
    w2f = Wl2 * s2[None, :]
    c2 = (bl2 * s2 + bn2_b).reshape(1, D2)

    w_out, fo_out = _sc_gather(w2oneT, w2mulT, w1cat, idx_all, rinv16)
    return _tc_mlp(w_out, fo_out, w1f, c1, w2f, c2, bias)
